# baseline (speedup 1.0000x reference)
import jax
import jax.numpy as jnp
from jax import lax
from jax.experimental import pallas as pl
from jax.experimental.pallas import tpu as pltpu
from jax.experimental.pallas import tpu_sc as plsc

N = 10000
E = 320000
D = 128
DE = 8
G = 100
NW = 50
NDF = 8
H1 = 16

NC = 2
NS = 16
CHUNK = 2048
NB = 5
PERW = E // (NC * NS)
TAIL = PERW - (NB - 1) * CHUNK
MSLAB = N // NS
NPAD = 10112
SLAB = NPAD // NS


def _relu(v):
    return jnp.maximum(v, 0.0)


BA = 1000


def _stage_a_body(x_ref, wp1, bp1, wp2, bp2, wp3, bp3,
                  wq1, bq1, wq2, bq2, wq3, bq3, wn1, ws1,
                  xprep_ref, msg_ref, xwn_ref, xws_ref, dagf_ref):
    x = x_ref[...]
    wx = jnp.concatenate([wp1[...], wn1[0:D], ws1[0:D]], axis=1)
    hx = x @ wx
    h = _relu(hx[:, 0:H1] + bp1[...])
    h = _relu(h @ wp2[...] + bp2[...])
    xp = h @ wp3[...] + bp3[...]
    xprep_ref[...] = xp
    m = _relu(xp @ wq1[...] + bq1[...])
    m = _relu(m @ wq2[...] + bq2[...])
    m = m @ wq3[...] + bq3[...]
    msg_ref[...] = jnp.pad(m, ((0, 0), (0, 16 - DE)))
    xwn_ref[...] = hx[:, H1:2 * H1]
    xws_ref[...] = hx[:, 2 * H1:3 * H1]
    i = pl.program_id(0)
    nd = G * BA // N
    dagf_ref[pl.ds(i * nd, nd), :] = x.reshape(-1, N // G, D)[:, 0, 0:NDF]


def _full_spec(a):
    nd = a.ndim
    return pl.BlockSpec(a.shape, lambda i, _n=nd: (0,) * _n)


def _stage_a(x, params):
    pp, pq = params["prep"], params["proc"]
    out_shape = (
        jax.ShapeDtypeStruct((N, DE), jnp.float32),
        jax.ShapeDtypeStruct((N, 16), jnp.float32),
        jax.ShapeDtypeStruct((N, H1), jnp.float32),
        jax.ShapeDtypeStruct((N, H1), jnp.float32),
        jax.ShapeDtypeStruct((G, NDF), jnp.float32),
    )
    args = (x, pp["W1"], pp["b1"], pp["W2"], pp["b2"], pp["W3"], pp["b3"],
            pq["W1"], pq["b1"], pq["W2"], pq["b2"], pq["W3"], pq["b3"],
            params["node"]["W1"], params["node_score"]["W1"])
    in_specs = [pl.BlockSpec((BA, D), lambda i: (i, 0))] + [
        _full_spec(a) for a in args[1:]]
    return pl.pallas_call(
        _stage_a_body,
        grid=(N // BA,),
        in_specs=in_specs,
        out_specs=(pl.BlockSpec((BA, DE), lambda i: (i, 0)),
                   pl.BlockSpec((BA, 16), lambda i: (i, 0)),
                   pl.BlockSpec((BA, H1), lambda i: (i, 0)),
                   pl.BlockSpec((BA, H1), lambda i: (i, 0)),
                   pl.BlockSpec((G, NDF), lambda i: (0, 0))),
        out_shape=out_shape,
    )(*args)


def _stage_b_body(msg_hbm, edge_hbm, zero_hbm, out_hbm,
                  agg_sh, msg_sh, src_v, dst_v, rows_a, rows_b,
                  sem_i, sem_a, sem_b):
    cid = lax.axis_index("c")
    sid = lax.axis_index("s")
    wid = cid * NS + sid
    base = wid * PERW
    pltpu.async_copy(msg_hbm.at[pl.ds(sid * MSLAB, MSLAB)],
                     msg_sh.at[pl.ds(sid * MSLAB, MSLAB)], sem_i)
    pltpu.sync_copy(zero_hbm, agg_sh.at[pl.ds(sid * SLAB, SLAB)])
    for j in range(NB - 1):
        pltpu.async_copy(
            edge_hbm.at[pl.ds(base + j * CHUNK, CHUNK)], src_v.at[j], sem_i)
        pltpu.async_copy(
            edge_hbm.at[pl.ds(E + base + j * CHUNK, CHUNK)], dst_v.at[j],
            sem_i)
    pltpu.async_copy(edge_hbm.at[pl.ds(base + (NB - 1) * CHUNK, TAIL)],
                     src_v.at[NB - 1, pl.ds(0, TAIL)], sem_i)
    pltpu.async_copy(edge_hbm.at[pl.ds(E + base + (NB - 1) * CHUNK, TAIL)],
                     dst_v.at[NB - 1, pl.ds(0, TAIL)], sem_i)

    @pl.loop(0, CHUNK - TAIL, step=16)
    def _(i):
        src_v[NB - 1, pl.ds(TAIL + i, 16)] = jnp.zeros((16,), jnp.int32)
        dst_v[NB - 1, pl.ds(TAIL + i, 16)] = jnp.full((16,), N, jnp.int32)

    pltpu.make_async_copy(msg_hbm.at[pl.ds(sid * MSLAB, MSLAB)],
                          msg_sh.at[pl.ds(sid * MSLAB, MSLAB)], sem_i).wait()
    for j in range(NB - 1):
        pltpu.make_async_copy(
            edge_hbm.at[pl.ds(base + j * CHUNK, CHUNK)], src_v.at[j],
            sem_i).wait()
        pltpu.make_async_copy(
            edge_hbm.at[pl.ds(E + base + j * CHUNK, CHUNK)], dst_v.at[j],
            sem_i).wait()
    pltpu.make_async_copy(edge_hbm.at[pl.ds(base + (NB - 1) * CHUNK, TAIL)],
                          src_v.at[NB - 1, pl.ds(0, TAIL)], sem_i).wait()
    pltpu.make_async_copy(edge_hbm.at[pl.ds(E + base + (NB - 1) * CHUNK, TAIL)],
                          dst_v.at[NB - 1, pl.ds(0, TAIL)], sem_i).wait()
    plsc.subcore_barrier()

    pltpu.async_copy(msg_sh.at[src_v.at[0]], rows_a, sem_a)

    @pl.loop(0, NB - 1, step=2)
    def _(j):
        pltpu.async_copy(msg_sh.at[src_v.at[j + 1]], rows_b, sem_b)
        pltpu.make_async_copy(msg_sh.at[src_v.at[j]], rows_a, sem_a).wait()
        pltpu.sync_copy(rows_a, agg_sh.at[dst_v.at[j]], add=True)
        pltpu.async_copy(msg_sh.at[src_v.at[j + 2]], rows_a, sem_a)
        pltpu.make_async_copy(msg_sh.at[src_v.at[j + 1]], rows_b, sem_b).wait()
        pltpu.sync_copy(rows_b, agg_sh.at[dst_v.at[j + 1]], add=True)

    pltpu.make_async_copy(msg_sh.at[src_v.at[NB - 1]], rows_a, sem_a).wait()
    pltpu.sync_copy(rows_a, agg_sh.at[dst_v.at[NB - 1]], add=True)

    plsc.subcore_barrier()
    pltpu.sync_copy(agg_sh.at[pl.ds(sid * SLAB, SLAB)],
                    out_hbm.at[cid, pl.ds(sid * SLAB, SLAB)])


def _stage_b(msg_pad, edge_index, zero_slab):
    mesh = plsc.VectorSubcoreMesh(core_axis_name="c", subcore_axis_name="s")
    kern = pl.kernel(
        _stage_b_body,
        out_type=jax.ShapeDtypeStruct((NC, NPAD, 16), jnp.float32),
        mesh=mesh,
        scratch_types=[
            pltpu.VMEM_SHARED((NPAD, 16), jnp.float32),
            pltpu.VMEM_SHARED((N, 16), jnp.float32),
            pltpu.VMEM((NB, CHUNK), jnp.int32),
            pltpu.VMEM((NB, CHUNK), jnp.int32),
            pltpu.VMEM((CHUNK, 16), jnp.float32),
            pltpu.VMEM((CHUNK, 16), jnp.float32),
            pltpu.SemaphoreType.DMA,
            pltpu.SemaphoreType.DMA,
            pltpu.SemaphoreType.DMA,
        ],
        compiler_params=pltpu.CompilerParams(use_tc_tiling_on_sc=False),
    )
    return kern(msg_pad, edge_index, zero_slab)


def _stage_c_body(parts_ref, xprep_ref, xwn_ref, xws_ref, dagf_ref,
                  wa1, ba1, wa2, ba2, wa3, ba3,
                  wn1, bn1, wn2, bn2, wn3, bn3,
                  wd1, bd1, wd2, bd2, wd3, bd3,
                  ws1, bs1, ws2, bs2, ws3, bs3,
                  wf1, bf1, wf2, bf2, wf3, bf3,
                  nsc_ref, dsc_ref):
    agg = parts_ref[0, 0:N, 0:DE] + parts_ref[1, 0:N, 0:DE]
    ga = _relu(agg @ wa1[...] + ba1[...])
    ga = _relu(ga @ wa2[...] + ba2[...])
    node_emb = xprep_ref[...] + (ga @ wa3[...] + ba3[...])

    wne = jnp.concatenate([wn1[D:], ws1[D:D + DE]], axis=1)
    hne = node_emb @ wne
    h = _relu(xwn_ref[...] + hne[:, 0:H1] + bn1[...])
    h = _relu(h @ wn2[...] + bn2[...])
    nodes_merged = h @ wn3[...] + bn3[...]

    dag_emb = nodes_merged.reshape(G, N // G, DE).sum(axis=1)

    gd = _relu(dag_emb @ wd1[...] + bd1[...])
    gd = _relu(gd @ wd2[...] + bd2[...])
    gd = gd @ wd3[...] + bd3[...]
    glob = jnp.sum(gd, axis=0, keepdims=True)

    d1 = dag_emb @ ws1[D + DE:D + 2 * DE]
    drep = jnp.broadcast_to(d1[:, None, :],
                            (G, N // G, H1)).reshape(N, H1)
    s = _relu(xws_ref[...] + hne[:, H1:2 * H1] + drep
              + glob @ ws1[D + 2 * DE:] + bs1[...])
    s = _relu(s @ ws2[...] + bs2[...])
    nsc_ref[...] = s @ ws3[...] + bs3[...]

    m1 = dagf_ref[...] @ wf1[0:NDF] + dag_emb @ wf1[NDF:NDF + DE]
    g2 = glob @ wf1[NDF + DE:NDF + 2 * DE]
    w1 = (lax.broadcasted_iota(jnp.int32, (NW, 1), 0).astype(jnp.float32)
          @ wf1[NDF + 2 * DE:])
    pre = (m1[:, None, :] + w1[None, :, :] + g2 + bf1[...]
           ).reshape(G * NW, H1)
    hh = _relu(pre)
    hh = _relu(hh @ wf2[...] + bf2[...])
    dsc_ref[...] = hh @ wf3[...] + bf3[...]


def _stage_c(parts, x_prep, xwn, xws, dag_feats, params):
    pa, pn = params["agg"], params["node"]
    pd, ps, pf = params["dag"], params["node_score"], params["dag_score"]
    out_shape = (
        jax.ShapeDtypeStruct((N, 1), jnp.float32),
        jax.ShapeDtypeStruct((G * NW, 1), jnp.float32),
    )
    return pl.pallas_call(_stage_c_body, out_shape=out_shape)(
        parts, x_prep, xwn, xws, dag_feats,
        pa["W1"], pa["b1"], pa["W2"], pa["b2"], pa["W3"], pa["b3"],
        pn["W1"], pn["b1"], pn["W2"], pn["b2"], pn["W3"], pn["b3"],
        pd["W1"], pd["b1"], pd["W2"], pd["b2"], pd["W3"], pd["b3"],
        ps["W1"], ps["b1"], ps["W2"], ps["b2"], ps["W3"], ps["b3"],
        pf["W1"], pf["b1"], pf["W2"], pf["b2"], pf["W3"], pf["b3"])


@jax.jit
def kernel(x, edge_index, ptr, params):
    x_prep, msg_pad, xwn, xws, dag_feats = _stage_a(x, params)
    zero_slab = jnp.zeros((SLAB, 16), jnp.float32)
    parts = _stage_b(msg_pad, edge_index.reshape(2 * E), zero_slab)
    nsc, dsc = _stage_c(parts, x_prep, xwn, xws, dag_feats, params)
    return nsc[:, 0], dsc[:, 0].reshape(G, NW)


# speedup vs baseline: 1.0433x; 1.0433x over previous
import jax
import jax.numpy as jnp
from jax import lax
from jax.experimental import pallas as pl
from jax.experimental.pallas import tpu as pltpu
from jax.experimental.pallas import tpu_sc as plsc

N = 10000
E = 320000
D = 128
DE = 8
G = 100
NW = 50
NDF = 8
H1 = 16

NC = 2
NS = 16
CHUNK = 2048
NB = 5
PERW = E // (NC * NS)
TAIL = PERW - (NB - 1) * CHUNK
MSLAB = N // NS
NPAD = 10112
SLAB = NPAD // NS


def _relu(v):
    return jnp.maximum(v, 0.0)


BA = 2000


def _stage_a_body(x_ref, wp1, bp1, wp2, bp2, wp3, bp3,
                  wq1, bq1, wq2, bq2, wq3, bq3, wn1, ws1,
                  xprep_ref, msg_ref, xwn_ref, xws_ref, dagf_ref):
    x = x_ref[...]
    wx = jnp.concatenate([wp1[...], wn1[0:D], ws1[0:D]], axis=1)
    hx = x @ wx
    h = _relu(hx[:, 0:H1] + bp1[...])
    h = _relu(h @ wp2[...] + bp2[...])
    xp = h @ wp3[...] + bp3[...]
    xprep_ref[...] = xp
    m = _relu(xp @ wq1[...] + bq1[...])
    m = _relu(m @ wq2[...] + bq2[...])
    m = m @ wq3[...] + bq3[...]
    msg_ref[...] = jnp.pad(m, ((0, 0), (0, 16 - DE)))
    xwn_ref[...] = hx[:, H1:2 * H1]
    xws_ref[...] = hx[:, 2 * H1:3 * H1]
    i = pl.program_id(0)
    nd = G * BA // N
    dagf_ref[pl.ds(i * nd, nd), :] = x.reshape(-1, N // G, D)[:, 0, 0:NDF]


def _full_spec(a):
    nd = a.ndim
    return pl.BlockSpec(a.shape, lambda i, _n=nd: (0,) * _n)


def _stage_a(x, params):
    pp, pq = params["prep"], params["proc"]
    out_shape = (
        jax.ShapeDtypeStruct((N, DE), jnp.float32),
        jax.ShapeDtypeStruct((N, 16), jnp.float32),
        jax.ShapeDtypeStruct((N, H1), jnp.float32),
        jax.ShapeDtypeStruct((N, H1), jnp.float32),
        jax.ShapeDtypeStruct((G, NDF), jnp.float32),
    )
    args = (x, pp["W1"], pp["b1"], pp["W2"], pp["b2"], pp["W3"], pp["b3"],
            pq["W1"], pq["b1"], pq["W2"], pq["b2"], pq["W3"], pq["b3"],
            params["node"]["W1"], params["node_score"]["W1"])
    in_specs = [pl.BlockSpec((BA, D), lambda i: (i, 0))] + [
        _full_spec(a) for a in args[1:]]
    return pl.pallas_call(
        _stage_a_body,
        grid=(N // BA,),
        in_specs=in_specs,
        out_specs=(pl.BlockSpec((BA, DE), lambda i: (i, 0)),
                   pl.BlockSpec((BA, 16), lambda i: (i, 0)),
                   pl.BlockSpec((BA, H1), lambda i: (i, 0)),
                   pl.BlockSpec((BA, H1), lambda i: (i, 0)),
                   pl.BlockSpec((G, NDF), lambda i: (0, 0))),
        out_shape=out_shape,
    )(*args)


def _stage_b_body(msg_hbm, edge_hbm, zero_hbm, out_hbm,
                  agg_sh, msg_sh, src_v, dst_v, rows_a, rows_b,
                  sem_i, sem_a, sem_b):
    cid = lax.axis_index("c")
    sid = lax.axis_index("s")
    wid = cid * NS + sid
    base = wid * PERW
    pltpu.async_copy(msg_hbm.at[pl.ds(sid * MSLAB, MSLAB)],
                     msg_sh.at[pl.ds(sid * MSLAB, MSLAB)], sem_i)
    pltpu.sync_copy(zero_hbm, agg_sh.at[pl.ds(sid * SLAB, SLAB)])
    for j in range(NB - 1):
        pltpu.async_copy(
            edge_hbm.at[pl.ds(base + j * CHUNK, CHUNK)], src_v.at[j], sem_i)
        pltpu.async_copy(
            edge_hbm.at[pl.ds(E + base + j * CHUNK, CHUNK)], dst_v.at[j],
            sem_i)
    pltpu.async_copy(edge_hbm.at[pl.ds(base + (NB - 1) * CHUNK, TAIL)],
                     src_v.at[NB - 1, pl.ds(0, TAIL)], sem_i)
    pltpu.async_copy(edge_hbm.at[pl.ds(E + base + (NB - 1) * CHUNK, TAIL)],
                     dst_v.at[NB - 1, pl.ds(0, TAIL)], sem_i)

    @pl.loop(0, CHUNK - TAIL, step=16)
    def _(i):
        src_v[NB - 1, pl.ds(TAIL + i, 16)] = jnp.zeros((16,), jnp.int32)
        dst_v[NB - 1, pl.ds(TAIL + i, 16)] = jnp.full((16,), N, jnp.int32)

    pltpu.make_async_copy(msg_hbm.at[pl.ds(sid * MSLAB, MSLAB)],
                          msg_sh.at[pl.ds(sid * MSLAB, MSLAB)], sem_i).wait()
    for j in range(NB - 1):
        pltpu.make_async_copy(
            edge_hbm.at[pl.ds(base + j * CHUNK, CHUNK)], src_v.at[j],
            sem_i).wait()
        pltpu.make_async_copy(
            edge_hbm.at[pl.ds(E + base + j * CHUNK, CHUNK)], dst_v.at[j],
            sem_i).wait()
    pltpu.make_async_copy(edge_hbm.at[pl.ds(base + (NB - 1) * CHUNK, TAIL)],
                          src_v.at[NB - 1, pl.ds(0, TAIL)], sem_i).wait()
    pltpu.make_async_copy(edge_hbm.at[pl.ds(E + base + (NB - 1) * CHUNK, TAIL)],
                          dst_v.at[NB - 1, pl.ds(0, TAIL)], sem_i).wait()
    plsc.subcore_barrier()

    pltpu.async_copy(msg_sh.at[src_v.at[0]], rows_a, sem_a)

    @pl.loop(0, NB - 1, step=2)
    def _(j):
        pltpu.async_copy(msg_sh.at[src_v.at[j + 1]], rows_b, sem_b)
        pltpu.make_async_copy(msg_sh.at[src_v.at[j]], rows_a, sem_a).wait()
        pltpu.sync_copy(rows_a, agg_sh.at[dst_v.at[j]], add=True)
        pltpu.async_copy(msg_sh.at[src_v.at[j + 2]], rows_a, sem_a)
        pltpu.make_async_copy(msg_sh.at[src_v.at[j + 1]], rows_b, sem_b).wait()
        pltpu.sync_copy(rows_b, agg_sh.at[dst_v.at[j + 1]], add=True)

    pltpu.make_async_copy(msg_sh.at[src_v.at[NB - 1]], rows_a, sem_a).wait()
    pltpu.sync_copy(rows_a, agg_sh.at[dst_v.at[NB - 1]], add=True)

    plsc.subcore_barrier()
    pltpu.sync_copy(agg_sh.at[pl.ds(sid * SLAB, SLAB)],
                    out_hbm.at[cid, pl.ds(sid * SLAB, SLAB)])


def _stage_b(msg_pad, edge_index, zero_slab):
    mesh = plsc.VectorSubcoreMesh(core_axis_name="c", subcore_axis_name="s")
    kern = pl.kernel(
        _stage_b_body,
        out_type=jax.ShapeDtypeStruct((NC, NPAD, 16), jnp.float32),
        mesh=mesh,
        scratch_types=[
            pltpu.VMEM_SHARED((NPAD, 16), jnp.float32),
            pltpu.VMEM_SHARED((N, 16), jnp.float32),
            pltpu.VMEM((NB, CHUNK), jnp.int32),
            pltpu.VMEM((NB, CHUNK), jnp.int32),
            pltpu.VMEM((CHUNK, 16), jnp.float32),
            pltpu.VMEM((CHUNK, 16), jnp.float32),
            pltpu.SemaphoreType.DMA,
            pltpu.SemaphoreType.DMA,
            pltpu.SemaphoreType.DMA,
        ],
        compiler_params=pltpu.CompilerParams(use_tc_tiling_on_sc=False),
    )
    return kern(msg_pad, edge_index, zero_slab)


def _stage_c_body(parts_ref, xprep_ref, xwn_ref, xws_ref, dagf_ref,
                  wa1, ba1, wa2, ba2, wa3, ba3,
                  wn1, bn1, wn2, bn2, wn3, bn3,
                  wd1, bd1, wd2, bd2, wd3, bd3,
                  ws1, bs1, ws2, bs2, ws3, bs3,
                  wf1, bf1, wf2, bf2, wf3, bf3,
                  nsc_ref, dsc_ref):
    agg = parts_ref[0, 0:N, 0:DE] + parts_ref[1, 0:N, 0:DE]
    ga = _relu(agg @ wa1[...] + ba1[...])
    ga = _relu(ga @ wa2[...] + ba2[...])
    node_emb = xprep_ref[...] + (ga @ wa3[...] + ba3[...])

    wne = jnp.concatenate([wn1[D:], ws1[D:D + DE]], axis=1)
    hne = node_emb @ wne
    h = _relu(xwn_ref[...] + hne[:, 0:H1] + bn1[...])
    h = _relu(h @ wn2[...] + bn2[...])
    nodes_merged = h @ wn3[...] + bn3[...]

    dag_emb = nodes_merged.reshape(G, N // G, DE).sum(axis=1)

    gd = _relu(dag_emb @ wd1[...] + bd1[...])
    gd = _relu(gd @ wd2[...] + bd2[...])
    gd = gd @ wd3[...] + bd3[...]
    glob = jnp.sum(gd, axis=0, keepdims=True)

    d1 = dag_emb @ ws1[D + DE:D + 2 * DE]
    drep = jnp.broadcast_to(d1[:, None, :],
                            (G, N // G, H1)).reshape(N, H1)
    s = _relu(xws_ref[...] + hne[:, H1:2 * H1] + drep
              + glob @ ws1[D + 2 * DE:] + bs1[...])
    s = _relu(s @ ws2[...] + bs2[...])
    nsc_ref[...] = s @ ws3[...] + bs3[...]

    m1 = dagf_ref[...] @ wf1[0:NDF] + dag_emb @ wf1[NDF:NDF + DE]
    g2 = glob @ wf1[NDF + DE:NDF + 2 * DE]
    w1 = (lax.broadcasted_iota(jnp.int32, (NW, 1), 0).astype(jnp.float32)
          @ wf1[NDF + 2 * DE:])
    pre = (m1[:, None, :] + w1[None, :, :] + g2 + bf1[...]
           ).reshape(G * NW, H1)
    hh = _relu(pre)
    hh = _relu(hh @ wf2[...] + bf2[...])
    dsc_ref[...] = hh @ wf3[...] + bf3[...]


def _stage_c(parts, x_prep, xwn, xws, dag_feats, params):
    pa, pn = params["agg"], params["node"]
    pd, ps, pf = params["dag"], params["node_score"], params["dag_score"]
    out_shape = (
        jax.ShapeDtypeStruct((N, 1), jnp.float32),
        jax.ShapeDtypeStruct((G * NW, 1), jnp.float32),
    )
    return pl.pallas_call(_stage_c_body, out_shape=out_shape)(
        parts, x_prep, xwn, xws, dag_feats,
        pa["W1"], pa["b1"], pa["W2"], pa["b2"], pa["W3"], pa["b3"],
        pn["W1"], pn["b1"], pn["W2"], pn["b2"], pn["W3"], pn["b3"],
        pd["W1"], pd["b1"], pd["W2"], pd["b2"], pd["W3"], pd["b3"],
        ps["W1"], ps["b1"], ps["W2"], ps["b2"], ps["W3"], ps["b3"],
        pf["W1"], pf["b1"], pf["W2"], pf["b2"], pf["W3"], pf["b3"])


@jax.jit
def kernel(x, edge_index, ptr, params):
    x_prep, msg_pad, xwn, xws, dag_feats = _stage_a(x, params)
    zero_slab = jnp.zeros((SLAB, 16), jnp.float32)
    parts = _stage_b(msg_pad, edge_index.reshape(2 * E), zero_slab)
    nsc, dsc = _stage_c(parts, x_prep, xwn, xws, dag_feats, params)
    return nsc[:, 0], dsc[:, 0].reshape(G, NW)
